# Initial kernel scaffold; baseline (speedup 1.0000x reference)
import jax
import jax.numpy as jnp
from jax.experimental import pallas as pl


def kernel(x, edge_index, idx, W, b):
    raise NotImplementedError("write your pallas kernel here")


# speedup vs baseline: 8.4758x; 8.4758x over previous
import functools

import jax
import jax.numpy as jnp
from jax import lax
from jax.experimental import pallas as pl
from jax.experimental.pallas import tpu as pltpu
from jax.experimental.pallas import tpu_sc as plsc

N = 10000
D = 128
E = 320000
NIDX = 5000

NC, NS = 2, 16
NW = NC * NS
EPW = E // NW
CH = 80
NCHUNK = EPW // CH
NP = NW * 160
OPW = NP // NS
OCH = OPW // CH
RPT = 624


def _sc_aggregate(x, edge_index, idx_pad):
  mesh = plsc.VectorSubcoreMesh(core_axis_name="c", subcore_axis_name="s",
                                num_cores=NC, num_subcores=NS)

  @functools.partial(
      pl.kernel,
      out_type=[
          jax.ShapeDtypeStruct((NP, D), jnp.float32),
          jax.ShapeDtypeStruct((NP, D), jnp.float32),
          jax.ShapeDtypeStruct((NP,), jnp.float32),
          jax.ShapeDtypeStruct((NP,), jnp.float32),
      ],
      mesh=mesh,
      scratch_types=[
          pltpu.VMEM_SHARED((N, D), jnp.float32),
          pltpu.VMEM_SHARED((N,), jnp.float32),
          pltpu.VMEM((CH,), jnp.int32),
          pltpu.VMEM((CH,), jnp.int32),
          pltpu.VMEM((CH, D), jnp.float32),
          pltpu.VMEM((1008,), jnp.float32),
          pltpu.VMEM((1008,), jnp.float32),
          pltpu.VMEM((48, D), jnp.float32),
          pltpu.VMEM((CH,), jnp.int32),
          pltpu.VMEM((CH,), jnp.float32),
          pltpu.SemaphoreType.DMA,
      ],
  )
  def body(x_hbm, src_hbm, dst_hbm, idx_hbm, gs0, gs1, gc0, gc1,
           acc, cacc, srcc, dstc, rows, ones, czero, zrow, idxc, gcv, sem):
    c = lax.axis_index("c")
    s = lax.axis_index("s")
    wid = c * NS + s

    def fill_ones(i, _):
      ones[pl.ds(i * 16, 16)] = jnp.full((16,), 1.0, jnp.float32)
      return 0
    lax.fori_loop(0, 63, fill_ones, 0)

    @pl.when(c == 1)
    def _():
      def fill_cz(i, _):
        czero[pl.ds(i * 16, 16)] = jnp.zeros((16,), jnp.float32)
        return 0
      lax.fori_loop(0, 63, fill_cz, 0)

      def fill_zr(i, _):
        zrow[i // 8, pl.ds((i % 8) * 16, 16)] = jnp.zeros((16,), jnp.float32)
        return 0
      lax.fori_loop(0, 48 * 8, fill_zr, 0)

    rb = s * RPT
    @pl.when(c == 0)
    def _():
      pltpu.sync_copy(x_hbm.at[pl.ds(rb, RPT)], acc.at[pl.ds(rb, RPT)])
      @pl.when(s == NS - 1)
      def _():
        pltpu.sync_copy(x_hbm.at[pl.ds(NS * RPT, N - NS * RPT)],
                        acc.at[pl.ds(NS * RPT, N - NS * RPT)])

    @pl.when(c == 1)
    def _():
      def zc(i, _):
        pltpu.sync_copy(zrow, acc.at[pl.ds(rb + i * 48, 48)])
        return 0
      lax.fori_loop(0, RPT // 48, zc, 0)
      @pl.when(s == NS - 1)
      def _():
        pltpu.sync_copy(zrow.at[pl.ds(0, N - NS * RPT)],
                        acc.at[pl.ds(NS * RPT, N - NS * RPT)])

    @pl.when(s < 10)
    def _():
      @pl.when(c == 0)
      def _():
        pltpu.sync_copy(ones.at[pl.ds(0, 1000)], cacc.at[pl.ds(s * 1000, 1000)])
      @pl.when(c == 1)
      def _():
        pltpu.sync_copy(czero.at[pl.ds(0, 1000)], cacc.at[pl.ds(s * 1000, 1000)])

    plsc.subcore_barrier()

    ebase = wid * EPW
    def ebody(j, _):
      base = pl.multiple_of(ebase + j * CH, 8)
      pltpu.sync_copy(src_hbm.at[pl.ds(base, CH)], srcc)
      pltpu.sync_copy(dst_hbm.at[pl.ds(base, CH)], dstc)
      pltpu.async_copy(x_hbm.at[srcc], rows, sem).wait()
      pltpu.sync_copy(rows, acc.at[dstc], add=True)
      pltpu.sync_copy(ones.at[pl.ds(0, CH)], cacc.at[dstc], add=True)
      return 0
    lax.fori_loop(0, NCHUNK, ebody, 0)

    plsc.subcore_barrier()

    def gbody(q, _):
      ob = pl.multiple_of(s * OPW + q * CH, 8)
      pltpu.sync_copy(idx_hbm.at[pl.ds(ob, CH)], idxc)
      pltpu.async_copy(acc.at[idxc], rows, sem).wait()
      pltpu.async_copy(cacc.at[idxc], gcv, sem).wait()
      @pl.when(c == 0)
      def _():
        pltpu.sync_copy(rows, gs0.at[pl.ds(ob, CH)])
        pltpu.sync_copy(gcv, gc0.at[pl.ds(ob, CH)])
      @pl.when(c == 1)
      def _():
        pltpu.sync_copy(rows, gs1.at[pl.ds(ob, CH)])
        pltpu.sync_copy(gcv, gc1.at[pl.ds(ob, CH)])
      return 0
    lax.fori_loop(0, OCH, gbody, 0)

  return body(x, edge_index[0], edge_index[1], idx_pad)


def _tc_finalize(gs0, gs1, gc0, gc1, W, b2):
  BR = 640

  def body(g0_ref, g1_ref, c0_ref, c1_ref, w_ref, b_ref, o_ref):
    g = g0_ref[...] + g1_ref[...]
    cnt = c0_ref[...] + c1_ref[...]
    m = g / cnt
    o_ref[...] = lax.dot_general(
        m, w_ref[...], (((1,), (1,)), ((), ())),
        preferred_element_type=jnp.float32) + b_ref[...]

  return pl.pallas_call(
      body,
      grid=(NP // BR,),
      in_specs=[
          pl.BlockSpec((BR, D), lambda i: (i, 0)),
          pl.BlockSpec((BR, D), lambda i: (i, 0)),
          pl.BlockSpec((BR, 1), lambda i: (i, 0)),
          pl.BlockSpec((BR, 1), lambda i: (i, 0)),
          pl.BlockSpec((D, D), lambda i: (0, 0)),
          pl.BlockSpec((1, D), lambda i: (0, 0)),
      ],
      out_specs=pl.BlockSpec((BR, D), lambda i: (i, 0)),
      out_shape=jax.ShapeDtypeStruct((NP, D), jnp.float32),
  )(gs0, gs1, gc0, gc1, W, b2)


def kernel(x, edge_index, idx, W, b):
  ei = edge_index.astype(jnp.int32)
  idx32 = idx.astype(jnp.int32)
  idx_pad = jnp.concatenate([idx32, jnp.zeros((NP - NIDX,), jnp.int32)])
  gs0, gs1, gc0, gc1 = _sc_aggregate(x, ei, idx_pad)
  out_full = _tc_finalize(gs0, gs1, gc0.reshape(NP, 1), gc1.reshape(NP, 1),
                          W, b.reshape(1, D))
  return out_full[:NIDX], idx


# speedup vs baseline: 15.1822x; 1.7912x over previous
import functools

import jax
import jax.numpy as jnp
from jax import lax
from jax.experimental import pallas as pl
from jax.experimental.pallas import tpu as pltpu
from jax.experimental.pallas import tpu_sc as plsc

N = 10000
D = 128
E = 320000
NIDX = 5000

NC, NS = 2, 16
NW = NC * NS
EPW = E // NW
CH = 80
NCHUNK = EPW // CH
NP = NW * 160
OPW = NP // NS
OCH = OPW // CH
RPT = 624


def _sc_aggregate(x, src, dst, idx_pad):
  mesh = plsc.VectorSubcoreMesh(core_axis_name="c", subcore_axis_name="s",
                                num_cores=NC, num_subcores=NS)

  @functools.partial(
      pl.kernel,
      out_type=[
          jax.ShapeDtypeStruct((NP, D), jnp.float32),
          jax.ShapeDtypeStruct((NP, D), jnp.float32),
          jax.ShapeDtypeStruct((NP,), jnp.float32),
          jax.ShapeDtypeStruct((NP,), jnp.float32),
      ],
      mesh=mesh,
      scratch_types=[
          pltpu.VMEM_SHARED((N, D), jnp.float32),
          pltpu.VMEM_SHARED((N,), jnp.float32),
          pltpu.VMEM((2, CH), jnp.int32),
          pltpu.VMEM((2, CH), jnp.int32),
          pltpu.VMEM((2, CH, D), jnp.float32),
          pltpu.VMEM((1008,), jnp.float32),
          pltpu.VMEM((1008,), jnp.float32),
          pltpu.VMEM((48, D), jnp.float32),
          pltpu.VMEM((CH,), jnp.int32),
          pltpu.VMEM((CH,), jnp.float32),
          pltpu.SemaphoreType.DMA,
          pltpu.SemaphoreType.DMA,
          pltpu.SemaphoreType.DMA,
      ],
  )
  def body(x_hbm, src_hbm, dst_hbm, idx_hbm, gs0, gs1, gc0, gc1,
           acc, cacc, srcc, dstc, rows, ones, czero, zrow, idxc, gcv,
           sem_g, sem_i, sem_c):
    c = lax.axis_index("c")
    s = lax.axis_index("s")
    wid = c * NS + s

    def fill_ones(i, _):
      ones[pl.ds(i * 16, 16)] = jnp.full((16,), 1.0, jnp.float32)
      return 0
    lax.fori_loop(0, 63, fill_ones, 0)

    @pl.when(c == 1)
    def _():
      def fill_cz(i, _):
        czero[pl.ds(i * 16, 16)] = jnp.zeros((16,), jnp.float32)
        return 0
      lax.fori_loop(0, 63, fill_cz, 0)

      def fill_zr(i, _):
        zrow[i // 8, pl.ds((i % 8) * 16, 16)] = jnp.zeros((16,), jnp.float32)
        return 0
      lax.fori_loop(0, 48 * 8, fill_zr, 0)

    rb = s * RPT
    @pl.when(c == 0)
    def _():
      pltpu.sync_copy(x_hbm.at[pl.ds(rb, RPT)], acc.at[pl.ds(rb, RPT)])
      @pl.when(s == NS - 1)
      def _():
        pltpu.sync_copy(x_hbm.at[pl.ds(NS * RPT, N - NS * RPT)],
                        acc.at[pl.ds(NS * RPT, N - NS * RPT)])

    @pl.when(c == 1)
    def _():
      def zc(i, _):
        pltpu.sync_copy(zrow, acc.at[pl.ds(rb + i * 48, 48)])
        return 0
      lax.fori_loop(0, RPT // 48, zc, 0)
      @pl.when(s == NS - 1)
      def _():
        pltpu.sync_copy(zrow.at[pl.ds(0, N - NS * RPT)],
                        acc.at[pl.ds(NS * RPT, N - NS * RPT)])

    @pl.when(s < 10)
    def _():
      @pl.when(c == 0)
      def _():
        pltpu.sync_copy(ones.at[pl.ds(0, 1000)], cacc.at[pl.ds(s * 1000, 1000)])
      @pl.when(c == 1)
      def _():
        pltpu.sync_copy(czero.at[pl.ds(0, 1000)], cacc.at[pl.ds(s * 1000, 1000)])

    plsc.subcore_barrier()

    ebase = wid * EPW

    b0 = pl.multiple_of(ebase, 8)
    pltpu.async_copy(src_hbm.at[pl.ds(b0, CH)], srcc.at[0], sem_i)
    pltpu.async_copy(dst_hbm.at[pl.ds(b0, CH)], dstc.at[0], sem_i)
    pltpu.make_async_copy(src_hbm.at[pl.ds(b0, CH)], srcc.at[0], sem_i).wait()
    pltpu.make_async_copy(dst_hbm.at[pl.ds(b0, CH)], dstc.at[0], sem_i).wait()
    pltpu.async_copy(x_hbm.at[srcc.at[0]], rows.at[0], sem_g)

    def ebody(j, _):
      p = j % 2
      q = 1 - p
      @pl.when(j > 0)
      def _():
        pltpu.make_async_copy(gc0.at[pl.ds(0, CH)], gcv, sem_c).wait()
      @pl.when(j < NCHUNK - 1)
      def _():
        bn = pl.multiple_of(ebase + (j + 1) * CH, 8)
        pltpu.async_copy(src_hbm.at[pl.ds(bn, CH)], srcc.at[q], sem_i)
        pltpu.async_copy(dst_hbm.at[pl.ds(bn, CH)], dstc.at[q], sem_i)
      pltpu.make_async_copy(x_hbm.at[srcc.at[p]], rows.at[p], sem_g).wait()
      @pl.when(j < NCHUNK - 1)
      def _():
        pltpu.make_async_copy(src_hbm.at[pl.ds(0, CH)], srcc.at[q], sem_i).wait()
        pltpu.make_async_copy(dst_hbm.at[pl.ds(0, CH)], dstc.at[q], sem_i).wait()
        pltpu.async_copy(x_hbm.at[srcc.at[q]], rows.at[q], sem_g)
      pltpu.sync_copy(rows.at[p], acc.at[dstc.at[p]], add=True)
      pltpu.async_copy(ones.at[pl.ds(0, CH)], cacc.at[dstc.at[p]], sem_c,
                       add=True)
      return 0
    lax.fori_loop(0, NCHUNK, ebody, 0)
    pltpu.make_async_copy(gc0.at[pl.ds(0, CH)], gcv, sem_c).wait()

    plsc.subcore_barrier()

    def gbody(qi, _):
      ob = pl.multiple_of(s * OPW + qi * CH, 8)
      pltpu.sync_copy(idx_hbm.at[pl.ds(ob, CH)], idxc)
      d1 = pltpu.async_copy(acc.at[idxc], rows.at[0], sem_g)
      d2 = pltpu.async_copy(cacc.at[idxc], gcv, sem_c)
      d1.wait()
      d2.wait()
      @pl.when(c == 0)
      def _():
        pltpu.sync_copy(rows.at[0], gs0.at[pl.ds(ob, CH)])
        pltpu.sync_copy(gcv, gc0.at[pl.ds(ob, CH)])
      @pl.when(c == 1)
      def _():
        pltpu.sync_copy(rows.at[0], gs1.at[pl.ds(ob, CH)])
        pltpu.sync_copy(gcv, gc1.at[pl.ds(ob, CH)])
      return 0
    lax.fori_loop(0, OCH, gbody, 0)

  return body(x, src, dst, idx_pad)


def _tc_finalize(gs0, gs1, gc0, gc1, W, b2):
  BR = 640

  def body(g0_ref, g1_ref, c0_ref, c1_ref, w_ref, b_ref, o_ref):
    g = g0_ref[...] + g1_ref[...]
    cnt = c0_ref[...] + c1_ref[...]
    m = g / cnt
    o_ref[...] = lax.dot_general(
        m, w_ref[...], (((1,), (1,)), ((), ())),
        preferred_element_type=jnp.float32) + b_ref[...]

  return pl.pallas_call(
      body,
      grid=(NP // BR,),
      in_specs=[
          pl.BlockSpec((BR, D), lambda i: (i, 0)),
          pl.BlockSpec((BR, D), lambda i: (i, 0)),
          pl.BlockSpec((BR, 1), lambda i: (i, 0)),
          pl.BlockSpec((BR, 1), lambda i: (i, 0)),
          pl.BlockSpec((D, D), lambda i: (0, 0)),
          pl.BlockSpec((1, D), lambda i: (0, 0)),
      ],
      out_specs=pl.BlockSpec((BR, D), lambda i: (i, 0)),
      out_shape=jax.ShapeDtypeStruct((NP, D), jnp.float32),
  )(gs0, gs1, gc0, gc1, W, b2)


def kernel(x, edge_index, idx, W, b):
  ei = edge_index.astype(jnp.int32)
  idx32 = idx.astype(jnp.int32)
  idx_pad = jnp.concatenate([idx32, jnp.zeros((NP - NIDX,), jnp.int32)])
  gs0, gs1, gc0, gc1 = _sc_aggregate(x, ei[0], ei[1], idx_pad)
  out_full = _tc_finalize(gs0, gs1, gc0.reshape(NP, 1), gc1.reshape(NP, 1),
                          W, b.reshape(1, D))
  return out_full[:NIDX], idx


# speedup vs baseline: 22.8669x; 1.5062x over previous
import functools

import jax
import jax.numpy as jnp
from jax import lax
from jax.experimental import pallas as pl
from jax.experimental.pallas import tpu as pltpu
from jax.experimental.pallas import tpu_sc as plsc

N = 10000
D = 128
E = 320000
NIDX = 5000

NC, NS = 2, 16
NW = NC * NS
EPW = E // NW
CH = 80
NP = NW * 160
OPW = NP // NS
OCH = OPW // CH
NA = NP + 8


def _sc_aggregate(x, src, dst, idx_pad):
  mesh = plsc.VectorSubcoreMesh(core_axis_name="c", subcore_axis_name="s",
                                num_cores=NC, num_subcores=NS)

  @functools.partial(
      pl.kernel,
      out_type=[
          jax.ShapeDtypeStruct((NP, D), jnp.float32),
          jax.ShapeDtypeStruct((NP, D), jnp.float32),
          jax.ShapeDtypeStruct((NP,), jnp.float32),
          jax.ShapeDtypeStruct((NP,), jnp.float32),
      ],
      mesh=mesh,
      compiler_params=pltpu.CompilerParams(needs_layout_passes=False),
      scratch_types=[
          pltpu.VMEM_SHARED((NA, D), jnp.float32),
          pltpu.VMEM_SHARED((NA,), jnp.float32),
          pltpu.VMEM((N,), jnp.int32),
          pltpu.VMEM((NP,), jnp.int32),
          pltpu.VMEM((EPW,), jnp.int32),
          pltpu.VMEM((EPW,), jnp.int32),
          pltpu.VMEM((EPW + CH,), jnp.int32),
          pltpu.VMEM((EPW + CH,), jnp.int32),
          pltpu.VMEM((2, CH), jnp.int32),
          pltpu.VMEM((2, CH, D), jnp.float32),
          pltpu.VMEM((1008,), jnp.float32),
          pltpu.VMEM((16, D), jnp.float32),
          pltpu.VMEM((CH,), jnp.int32),
          pltpu.VMEM((CH,), jnp.int32),
          pltpu.VMEM((CH,), jnp.float32),
          pltpu.SemaphoreType.DMA,
          pltpu.SemaphoreType.DMA,
      ],
  )
  def body(x_hbm, src_hbm, dst_hbm, idx_hbm, gs0, gs1, gc0, gc1,
           acc, cacc, mask, idxall, srcall, dstall, srcf, dstf, dstc2, rows,
           ones, zrow, idxc, slotc, gcv, sem_g, sem_c):
    c = lax.axis_index("c")
    s = lax.axis_index("s")
    wid = c * NS + s
    lane = lax.iota(jnp.int32, 16)
    trash16 = NP + (lane & 7)

    ebase = pl.multiple_of(wid * EPW, 8)
    pltpu.async_copy(src_hbm.at[pl.ds(ebase, EPW)], srcall, sem_g)
    pltpu.async_copy(dst_hbm.at[pl.ds(ebase, EPW)], dstall, sem_g)
    pltpu.async_copy(idx_hbm, idxall, sem_g)

    def fill_ones(i, _):
      ones[pl.ds(i * 16, 16)] = jnp.full((16,), 1.0, jnp.float32)
      return 0
    lax.fori_loop(0, 63, fill_ones, 0)

    @pl.when(c == 1)
    def _():
      def fill_zr(i, _):
        zrow[i // 8, pl.ds((i % 8) * 16, 16)] = jnp.zeros((16,), jnp.float32)
        return 0
      lax.fori_loop(0, 16 * 8, fill_zr, 0)

    def zm(i, _):
      mask[pl.ds(i * 16, 16)] = trash16
      return 0
    lax.fori_loop(0, N // 16, zm, 0)

    @pl.when(s < 5)
    def _():
      pltpu.sync_copy(ones.at[pl.ds(0, 1008)], cacc.at[pl.ds(s * 1008, 1008)])
    @pl.when(s == 5)
    def _():
      pltpu.sync_copy(ones.at[pl.ds(0, NA - 5 * 1008)],
                      cacc.at[pl.ds(5 * 1008, NA - 5 * 1008)])

    @pl.when(c == 1)
    def _():
      def zc(i, _):
        pltpu.sync_copy(zrow, acc.at[pl.ds(s * 320 + i * 16, 16)])
        return 0
      lax.fori_loop(0, 20, zc, 0)
      @pl.when(s == NS - 1)
      def _():
        pltpu.sync_copy(zrow.at[pl.ds(0, 8)], acc.at[pl.ds(NP, 8)])

    pltpu.make_async_copy(src_hbm.at[pl.ds(ebase, EPW)], srcall, sem_g).wait()
    pltpu.make_async_copy(dst_hbm.at[pl.ds(ebase, EPW)], dstall, sem_g).wait()
    pltpu.make_async_copy(idx_hbm, idxall, sem_g).wait()

    def bm(g, _):
      iv = idxall[pl.ds(g * 16, 16)]
      plsc.store_scatter(mask, [iv], g * 16 + lane)
      return 0
    lax.fori_loop(0, NP // 16, bm, 0)

    @pl.when(c == 0)
    def _():
      def xi(qi, _):
        ob = pl.multiple_of(s * OPW + qi * CH, 8)
        def cpi(g, _):
          idxc[pl.ds(g * 16, 16)] = idxall[pl.ds(ob + g * 16, 16)]
          return 0
        lax.fori_loop(0, CH // 16, cpi, 0)
        pltpu.async_copy(x_hbm.at[idxc], rows.at[0], sem_g).wait()
        pltpu.sync_copy(rows.at[0], acc.at[pl.ds(ob, CH)])
        return 0
      lax.fori_loop(0, OCH, xi, 0)

    def fb(g, off):
      sval = srcall[pl.ds(g * 16, 16)]
      dval = dstall[pl.ds(g * 16, 16)]
      mv = plsc.load_gather(mask, [dval])
      keep = mv < NP
      plsc.store_compressed(srcf.at[pl.ds(off, 16)], sval, mask=keep)
      plsc.store_compressed(dstf.at[pl.ds(off, 16)], mv, mask=keep)
      return off + plsc.all_reduce_population_count(keep)[0]
    off = lax.fori_loop(0, EPW // 16, fb, jnp.int32(0))

    nch = (off + (CH - 1)) // CH
    strash = lane & 7
    def pb(o):
      srcf[pl.ds(o, 16)] = strash
      dstf[pl.ds(o, 16)] = trash16
      return o + 16
    lax.while_loop(lambda o: o < nch * CH, pb, off)

    plsc.subcore_barrier()

    def cpd(q2, b2):
      def cg(g, _):
        dstc2[q2, pl.ds(g * 16, 16)] = dstf[pl.ds(b2 + g * 16, 16)]
        return 0
      lax.fori_loop(0, CH // 16, cg, 0)

    @pl.when(nch > 0)
    def _():
      cpd(0, jnp.int32(0))
      pltpu.async_copy(x_hbm.at[srcf.at[pl.ds(0, CH)]], rows.at[0], sem_g)

    def ebody(j, _):
      p = j % 2
      q = 1 - p
      @pl.when(j > 0)
      def _():
        pltpu.make_async_copy(gc0.at[pl.ds(0, CH)], gcv, sem_c).wait()
      @pl.when(j < nch - 1)
      def _():
        cpd(q, (j + 1) * CH)
      pltpu.make_async_copy(x_hbm.at[srcf.at[pl.ds(0, CH)]], rows.at[p],
                            sem_g).wait()
      @pl.when(j < nch - 1)
      def _():
        bn = pl.multiple_of((j + 1) * CH, 8)
        pltpu.async_copy(x_hbm.at[srcf.at[pl.ds(bn, CH)]], rows.at[q], sem_g)
      pltpu.sync_copy(rows.at[p], acc.at[dstc2.at[p]], add=True)
      pltpu.async_copy(ones.at[pl.ds(0, CH)], cacc.at[dstc2.at[p]], sem_c,
                       add=True)
      return 0
    lax.fori_loop(0, nch, ebody, 0)
    @pl.when(nch > 0)
    def _():
      pltpu.make_async_copy(gc0.at[pl.ds(0, CH)], gcv, sem_c).wait()

    plsc.subcore_barrier()

    def gbody(qi, _):
      ob = pl.multiple_of(s * OPW + qi * CH, 8)
      def sl(g, _):
        iv = idxall[pl.ds(ob + g * 16, 16)]
        slotc[pl.ds(g * 16, 16)] = plsc.load_gather(mask, [iv])
        return 0
      lax.fori_loop(0, CH // 16, sl, 0)
      d1 = pltpu.async_copy(acc.at[slotc], rows.at[0], sem_g)
      d2 = pltpu.async_copy(cacc.at[slotc], gcv, sem_c)
      d1.wait()
      d2.wait()
      @pl.when(c == 0)
      def _():
        pltpu.sync_copy(rows.at[0], gs0.at[pl.ds(ob, CH)])
        pltpu.sync_copy(gcv, gc0.at[pl.ds(ob, CH)])
      @pl.when(c == 1)
      def _():
        pltpu.sync_copy(rows.at[0], gs1.at[pl.ds(ob, CH)])
        pltpu.sync_copy(gcv, gc1.at[pl.ds(ob, CH)])
      return 0
    lax.fori_loop(0, OCH, gbody, 0)

  return body(x, src, dst, idx_pad)


def _tc_finalize(gs0, gs1, gc0, gc1, W, b2):
  BR = 640

  def body(g0_ref, g1_ref, c0_ref, c1_ref, w_ref, b_ref, o_ref):
    g = g0_ref[...] + g1_ref[...]
    cnt = c0_ref[...] + c1_ref[...] - 1.0
    m = g / cnt
    o_ref[...] = lax.dot_general(
        m, w_ref[...], (((1,), (1,)), ((), ())),
        preferred_element_type=jnp.float32) + b_ref[...]

  return pl.pallas_call(
      body,
      grid=(NP // BR,),
      in_specs=[
          pl.BlockSpec((BR, D), lambda i: (i, 0)),
          pl.BlockSpec((BR, D), lambda i: (i, 0)),
          pl.BlockSpec((BR, 1), lambda i: (i, 0)),
          pl.BlockSpec((BR, 1), lambda i: (i, 0)),
          pl.BlockSpec((D, D), lambda i: (0, 0)),
          pl.BlockSpec((1, D), lambda i: (0, 0)),
      ],
      out_specs=pl.BlockSpec((BR, D), lambda i: (i, 0)),
      out_shape=jax.ShapeDtypeStruct((NP, D), jnp.float32),
  )(gs0, gs1, gc0, gc1, W, b2)


def kernel(x, edge_index, idx, W, b):
  ei = edge_index.astype(jnp.int32)
  idx32 = idx.astype(jnp.int32)
  idx_pad = jnp.concatenate([idx32, jnp.zeros((NP - NIDX,), jnp.int32)])
  gs0, gs1, gc0, gc1 = _sc_aggregate(x, ei[0], ei[1], idx_pad)
  out_full = _tc_finalize(gs0, gs1, gc0.reshape(NP, 1), gc1.reshape(NP, 1),
                          W, b.reshape(1, D))
  return out_full[:NIDX], idx


# speedup vs baseline: 23.7879x; 1.0403x over previous
import functools

import jax
import jax.numpy as jnp
from jax import lax
from jax.experimental import pallas as pl
from jax.experimental.pallas import tpu as pltpu
from jax.experimental.pallas import tpu_sc as plsc

N = 10000
D = 128
E = 320000
NIDX = 5000

NC, NS = 2, 16
NW = NC * NS
EPW = E // NW
CH = 80
NP = NW * 160
OPW = NP // NS
OCH = OPW // CH
NA = NP + 8


def _sc_aggregate(x, src, dst, idx_pad):
  mesh = plsc.VectorSubcoreMesh(core_axis_name="c", subcore_axis_name="s",
                                num_cores=NC, num_subcores=NS)

  @functools.partial(
      pl.kernel,
      out_type=[
          jax.ShapeDtypeStruct((NP, D), jnp.float32),
          jax.ShapeDtypeStruct((NP, D), jnp.float32),
          jax.ShapeDtypeStruct((NP,), jnp.float32),
          jax.ShapeDtypeStruct((NP,), jnp.float32),
      ],
      mesh=mesh,
      compiler_params=pltpu.CompilerParams(needs_layout_passes=False),
      scratch_types=[
          pltpu.VMEM_SHARED((NA, D), jnp.float32),
          pltpu.VMEM_SHARED((NA,), jnp.float32),
          pltpu.VMEM((N,), jnp.int32),
          pltpu.VMEM((NP,), jnp.int32),
          pltpu.VMEM((EPW,), jnp.int32),
          pltpu.VMEM((EPW,), jnp.int32),
          pltpu.VMEM((EPW + CH,), jnp.int32),
          pltpu.VMEM((EPW + CH,), jnp.int32),
          pltpu.VMEM((2, CH), jnp.int32),
          pltpu.VMEM((2, CH, D), jnp.float32),
          pltpu.VMEM((1008,), jnp.float32),
          pltpu.VMEM((16, D), jnp.float32),
          pltpu.VMEM((CH,), jnp.int32),
          pltpu.VMEM((CH,), jnp.int32),
          pltpu.VMEM((CH,), jnp.float32),
          pltpu.SemaphoreType.DMA,
          pltpu.SemaphoreType.DMA,
          pltpu.SemaphoreType.DMA,
      ],
  )
  def body(x_hbm, src_hbm, dst_hbm, idx_hbm, gs0, gs1, gc0, gc1,
           acc, cacc, mask, idxall, srcall, dstall, srcf, dstf, dstc2, rows,
           ones, zrow, idxc, slotc, gcv, sem_g, sem_s, sem_c):
    c = lax.axis_index("c")
    s = lax.axis_index("s")
    wid = c * NS + s
    lane = lax.iota(jnp.int32, 16)
    trash16 = NP + (lane & 7)

    ebase = pl.multiple_of(wid * EPW, 8)
    pltpu.async_copy(src_hbm.at[pl.ds(ebase, EPW)], srcall, sem_g)
    pltpu.async_copy(dst_hbm.at[pl.ds(ebase, EPW)], dstall, sem_g)
    pltpu.async_copy(idx_hbm.at[pl.ds(0, NIDX - 8)], idxall.at[pl.ds(0, NIDX - 8)], sem_g)
    pltpu.async_copy(idx_hbm.at[pl.ds(NIDX - 16, 16)], idxall.at[pl.ds(NIDX - 16, 16)], sem_g)

    def fill_ones(i, _):
      ones[pl.ds(i * 16, 16)] = jnp.full((16,), 1.0, jnp.float32)
      return 0
    lax.fori_loop(0, 63, fill_ones, 0)

    @pl.when(c == 1)
    def _():
      def fill_zr(i, _):
        zrow[i // 8, pl.ds((i % 8) * 16, 16)] = jnp.zeros((16,), jnp.float32)
        return 0
      lax.fori_loop(0, 16 * 8, fill_zr, 0)

    def zm(i, _):
      mask[pl.ds(i * 16, 16)] = trash16
      return 0
    lax.fori_loop(0, N // 16, zm, 0)

    @pl.when(s < 5)
    def _():
      pltpu.sync_copy(ones.at[pl.ds(0, 1008)], cacc.at[pl.ds(s * 1008, 1008)])
    @pl.when(s == 5)
    def _():
      pltpu.sync_copy(ones.at[pl.ds(0, NA - 5 * 1008)],
                      cacc.at[pl.ds(5 * 1008, NA - 5 * 1008)])

    @pl.when(c == 1)
    def _():
      def zc(i, _):
        pltpu.sync_copy(zrow, acc.at[pl.ds(s * 320 + i * 16, 16)])
        return 0
      lax.fori_loop(0, 20, zc, 0)
      @pl.when(s == NS - 1)
      def _():
        pltpu.sync_copy(zrow.at[pl.ds(0, 8)], acc.at[pl.ds(NP, 8)])

    pltpu.make_async_copy(src_hbm.at[pl.ds(ebase, EPW)], srcall, sem_g).wait()
    pltpu.make_async_copy(dst_hbm.at[pl.ds(ebase, EPW)], dstall, sem_g).wait()
    pltpu.make_async_copy(idx_hbm.at[pl.ds(0, NIDX - 8)], idxall.at[pl.ds(0, NIDX - 8)], sem_g).wait()
    pltpu.make_async_copy(idx_hbm.at[pl.ds(NIDX - 16, 16)], idxall.at[pl.ds(NIDX - 16, 16)], sem_g).wait()
    tail = idxall[pl.ds(NIDX - 8, 16)]
    idxall[pl.ds(NIDX - 8, 16)] = jnp.where(lane < 8, tail, 0)
    def zpad(k, _):
      idxall[pl.ds(NIDX + 8 + k * 16, 16)] = jnp.zeros((16,), jnp.int32)
      return 0
    lax.fori_loop(0, (NP - NIDX - 8) // 16, zpad, 0)

    def bm(g, _):
      iv = idxall[pl.ds(g * 16, 16)]
      plsc.store_scatter(mask, [iv], g * 16 + lane)
      return 0
    lax.fori_loop(0, NP // 16, bm, 0)

    @pl.when(c == 0)
    def _():
      def xi(qi, _):
        ob = pl.multiple_of(s * OPW + qi * CH, 8)
        def cpi(g, _):
          idxc[pl.ds(g * 16, 16)] = idxall[pl.ds(ob + g * 16, 16)]
          return 0
        lax.fori_loop(0, CH // 16, cpi, 0)
        pltpu.async_copy(x_hbm.at[idxc], rows.at[0], sem_g).wait()
        pltpu.sync_copy(rows.at[0], acc.at[pl.ds(ob, CH)])
        return 0
      lax.fori_loop(0, OCH, xi, 0)

    def fb(g, off):
      sval = srcall[pl.ds(g * 16, 16)]
      dval = dstall[pl.ds(g * 16, 16)]
      mv = plsc.load_gather(mask, [dval])
      keep = mv < NP
      plsc.store_compressed(srcf.at[pl.ds(off, 16)], sval, mask=keep)
      plsc.store_compressed(dstf.at[pl.ds(off, 16)], mv, mask=keep)
      return off + plsc.all_reduce_population_count(keep)[0]
    off = lax.fori_loop(0, EPW // 16, fb, jnp.int32(0))

    nch = (off + (CH - 1)) // CH
    strash = lane & 7
    def pb(o):
      srcf[pl.ds(o, 16)] = strash
      dstf[pl.ds(o, 16)] = trash16
      return o + 16
    lax.while_loop(lambda o: o < nch * CH, pb, off)

    plsc.subcore_barrier()

    def cpd(q2, b2):
      def cg(g, _):
        dstc2[q2, pl.ds(g * 16, 16)] = dstf[pl.ds(b2 + g * 16, 16)]
        return 0
      lax.fori_loop(0, CH // 16, cg, 0)

    @pl.when(nch > 0)
    def _():
      cpd(0, jnp.int32(0))
      pltpu.async_copy(x_hbm.at[srcf.at[pl.ds(0, CH)]], rows.at[0], sem_g)

    def ebody(j, _):
      p = j % 2
      q = 1 - p
      @pl.when(j > 0)
      def _():
        pltpu.make_async_copy(rows.at[q], acc.at[dstc2.at[q]], sem_s).wait()
        pltpu.make_async_copy(gc0.at[pl.ds(0, CH)], gcv, sem_c).wait()
      @pl.when(j < nch - 1)
      def _():
        cpd(q, (j + 1) * CH)
      pltpu.make_async_copy(x_hbm.at[srcf.at[pl.ds(0, CH)]], rows.at[p],
                            sem_g).wait()
      @pl.when(j < nch - 1)
      def _():
        bn = pl.multiple_of((j + 1) * CH, 8)
        pltpu.async_copy(x_hbm.at[srcf.at[pl.ds(bn, CH)]], rows.at[q], sem_g)
      pltpu.async_copy(rows.at[p], acc.at[dstc2.at[p]], sem_s, add=True)
      pltpu.async_copy(ones.at[pl.ds(0, CH)], cacc.at[dstc2.at[p]], sem_c,
                       add=True)
      return 0
    lax.fori_loop(0, nch, ebody, 0)
    @pl.when(nch > 0)
    def _():
      p = (nch - 1) % 2
      pltpu.make_async_copy(rows.at[p], acc.at[dstc2.at[p]], sem_s).wait()
      pltpu.make_async_copy(gc0.at[pl.ds(0, CH)], gcv, sem_c).wait()

    plsc.subcore_barrier()

    def gbody(qi, _):
      ob = pl.multiple_of(s * OPW + qi * CH, 8)
      def sl(g, _):
        iv = idxall[pl.ds(ob + g * 16, 16)]
        slotc[pl.ds(g * 16, 16)] = plsc.load_gather(mask, [iv])
        return 0
      lax.fori_loop(0, CH // 16, sl, 0)
      d1 = pltpu.async_copy(acc.at[slotc], rows.at[0], sem_g)
      d2 = pltpu.async_copy(cacc.at[slotc], gcv, sem_c)
      d1.wait()
      d2.wait()
      @pl.when(c == 0)
      def _():
        pltpu.sync_copy(rows.at[0], gs0.at[pl.ds(ob, CH)])
        pltpu.sync_copy(gcv, gc0.at[pl.ds(ob, CH)])
      @pl.when(c == 1)
      def _():
        pltpu.sync_copy(rows.at[0], gs1.at[pl.ds(ob, CH)])
        pltpu.sync_copy(gcv, gc1.at[pl.ds(ob, CH)])
      return 0
    lax.fori_loop(0, OCH, gbody, 0)

  return body(x, src, dst, idx_pad)


def _tc_finalize(gs0, gs1, gc0, gc1, W, b2):
  BR = 1000

  def body(g0_ref, g1_ref, c0_ref, c1_ref, w_ref, b_ref, o_ref):
    g = g0_ref[...] + g1_ref[...]
    cnt = c0_ref[...] + c1_ref[...] - 1.0
    m = g / cnt
    o_ref[...] = lax.dot_general(
        m, w_ref[...], (((1,), (1,)), ((), ())),
        preferred_element_type=jnp.float32) + b_ref[...]

  return pl.pallas_call(
      body,
      grid=(NIDX // BR,),
      in_specs=[
          pl.BlockSpec((BR, D), lambda i: (i, 0)),
          pl.BlockSpec((BR, D), lambda i: (i, 0)),
          pl.BlockSpec((BR, 1), lambda i: (i, 0)),
          pl.BlockSpec((BR, 1), lambda i: (i, 0)),
          pl.BlockSpec((D, D), lambda i: (0, 0)),
          pl.BlockSpec((1, D), lambda i: (0, 0)),
      ],
      out_specs=pl.BlockSpec((BR, D), lambda i: (i, 0)),
      out_shape=jax.ShapeDtypeStruct((NIDX, D), jnp.float32),
  )(gs0, gs1, gc0, gc1, W, b2)


def kernel(x, edge_index, idx, W, b):
  ei = edge_index.astype(jnp.int32)
  idx32 = idx.astype(jnp.int32)
  gs0, gs1, gc0, gc1 = _sc_aggregate(x, ei[0], ei[1], idx32)
  out = _tc_finalize(gs0, gs1, gc0.reshape(NP, 1), gc1.reshape(NP, 1),
                     W, b.reshape(1, D))
  return out, idx


# speedup vs baseline: 24.1612x; 1.0157x over previous
import functools

import jax
import jax.numpy as jnp
from jax import lax
from jax.experimental import pallas as pl
from jax.experimental.pallas import tpu as pltpu
from jax.experimental.pallas import tpu_sc as plsc

N = 10000
D = 128
E = 320000
NIDX = 5000

NC, NS = 2, 16
NW = NC * NS
EPW = E // NW
CH = 80
NP = NW * 160
OPW = NP // NS
OCH = OPW // CH
NA = NP + 8


def _sc_aggregate(x, src, dst, idx_pad):
  mesh = plsc.VectorSubcoreMesh(core_axis_name="c", subcore_axis_name="s",
                                num_cores=NC, num_subcores=NS)

  @functools.partial(
      pl.kernel,
      out_type=[
          jax.ShapeDtypeStruct((NP, D), jnp.float32),
          jax.ShapeDtypeStruct((NP, D), jnp.float32),
          jax.ShapeDtypeStruct((NP,), jnp.float32),
          jax.ShapeDtypeStruct((NP,), jnp.float32),
      ],
      mesh=mesh,
      compiler_params=pltpu.CompilerParams(needs_layout_passes=False),
      scratch_types=[
          pltpu.VMEM_SHARED((NA, D), jnp.float32),
          pltpu.VMEM_SHARED((NA,), jnp.float32),
          pltpu.VMEM((N,), jnp.int32),
          pltpu.VMEM((NP,), jnp.int32),
          pltpu.VMEM((EPW,), jnp.int32),
          pltpu.VMEM((EPW,), jnp.int32),
          pltpu.VMEM((EPW + CH,), jnp.int32),
          pltpu.VMEM((EPW + CH,), jnp.int32),
          pltpu.VMEM((2, CH), jnp.int32),
          pltpu.VMEM((2, CH, D), jnp.float32),
          pltpu.VMEM((1008,), jnp.float32),
          pltpu.VMEM((16, D), jnp.float32),
          pltpu.VMEM((CH,), jnp.int32),
          pltpu.VMEM((CH,), jnp.int32),
          pltpu.VMEM((CH,), jnp.float32),
          pltpu.SemaphoreType.DMA,
          pltpu.SemaphoreType.DMA,
          pltpu.SemaphoreType.DMA,
      ],
  )
  def body(x_hbm, src_hbm, dst_hbm, idx_hbm, gs0, gs1, gc0, gc1,
           acc, cacc, mask, idxall, srcall, dstall, srcf, dstf, dstc2, rows,
           ones, zrow, idxc, slotc, gcv, sem_g, sem_s, sem_c):
    c = lax.axis_index("c")
    s = lax.axis_index("s")
    wid = c * NS + s
    lane = lax.iota(jnp.int32, 16)
    trash16 = NP + (lane & 7)

    ebase = pl.multiple_of(wid * EPW, 8)
    pltpu.async_copy(src_hbm.at[pl.ds(ebase, EPW)], srcall, sem_g)
    pltpu.async_copy(dst_hbm.at[pl.ds(ebase, EPW)], dstall, sem_g)
    pltpu.async_copy(idx_hbm.at[pl.ds(0, NIDX - 8)], idxall.at[pl.ds(0, NIDX - 8)], sem_g)
    pltpu.async_copy(idx_hbm.at[pl.ds(NIDX - 16, 16)], idxall.at[pl.ds(NIDX - 16, 16)], sem_g)

    def fill_ones(i, _):
      ones[pl.ds(i * 16, 16)] = jnp.full((16,), 1.0, jnp.float32)
      return 0
    lax.fori_loop(0, 63, fill_ones, 0)

    def fill_zr(i, _):
      zrow[i // 8, pl.ds((i % 8) * 16, 16)] = jnp.zeros((16,), jnp.float32)
      return 0
    lax.fori_loop(0, 16 * 8, fill_zr, 0)

    def zm(i, _):
      mask[pl.ds(i * 16, 16)] = trash16
      return 0
    lax.fori_loop(0, N // 16, zm, 0)

    @pl.when(s < 5)
    def _():
      pltpu.sync_copy(ones.at[pl.ds(0, 1008)], cacc.at[pl.ds(s * 1008, 1008)])
    @pl.when(s == 5)
    def _():
      pltpu.sync_copy(ones.at[pl.ds(0, NA - 5 * 1008)],
                      cacc.at[pl.ds(5 * 1008, NA - 5 * 1008)])

    zbase = s * OPW + (1 - c) * (OPW // 2)
    def zc(i, _):
      pltpu.sync_copy(zrow, acc.at[pl.ds(zbase + i * 16, 16)])
      return 0
    lax.fori_loop(0, (OPW // 2) // 16, zc, 0)
    @pl.when(s == NS - 1)
    def _():
      pltpu.sync_copy(zrow.at[pl.ds(0, 8)], acc.at[pl.ds(NP, 8)])

    pltpu.make_async_copy(src_hbm.at[pl.ds(ebase, EPW)], srcall, sem_g).wait()
    pltpu.make_async_copy(dst_hbm.at[pl.ds(ebase, EPW)], dstall, sem_g).wait()
    pltpu.make_async_copy(idx_hbm.at[pl.ds(0, NIDX - 8)], idxall.at[pl.ds(0, NIDX - 8)], sem_g).wait()
    pltpu.make_async_copy(idx_hbm.at[pl.ds(NIDX - 16, 16)], idxall.at[pl.ds(NIDX - 16, 16)], sem_g).wait()
    tail = idxall[pl.ds(NIDX - 8, 16)]
    idxall[pl.ds(NIDX - 8, 16)] = jnp.where(lane < 8, tail, 0)
    def zpad(k, _):
      idxall[pl.ds(NIDX + 8 + k * 16, 16)] = jnp.zeros((16,), jnp.int32)
      return 0
    lax.fori_loop(0, (NP - NIDX - 8) // 16, zpad, 0)

    def bm(g, _):
      iv = idxall[pl.ds(g * 16, 16)]
      plsc.store_scatter(mask, [iv], g * 16 + lane)
      return 0
    lax.fori_loop(0, NP // 16, bm, 0)

    def xi(qi, _):
      ob = pl.multiple_of(s * OPW + c * (OPW // 2) + qi * CH, 8)
      def cpi(g, _):
        idxc[pl.ds(g * 16, 16)] = idxall[pl.ds(ob + g * 16, 16)]
        return 0
      lax.fori_loop(0, CH // 16, cpi, 0)
      pltpu.async_copy(x_hbm.at[idxc], rows.at[0], sem_g).wait()
      pltpu.sync_copy(rows.at[0], acc.at[pl.ds(ob, CH)])
      return 0
    lax.fori_loop(0, (OPW // 2) // CH, xi, 0)

    def fb1(g, off):
      sval = srcall[pl.ds(g * 16, 16)]
      dval = dstall[pl.ds(g * 16, 16)]
      mv = plsc.load_gather(mask, [dval])
      keep = mv < NP
      plsc.store_compressed(srcf.at[pl.ds(off, 16)], sval, mask=keep)
      plsc.store_compressed(dstf.at[pl.ds(off, 16)], mv, mask=keep)
      return off + plsc.all_reduce_population_count(keep)[0]
    def fb2(h, off):
      return fb1(2 * h + 1, fb1(2 * h, off))
    off = lax.fori_loop(0, (EPW // 16) // 2, fb2, jnp.int32(0))
    off = fb1(EPW // 16 - 1, off)

    nch = (off + (CH - 1)) // CH
    strash = lane & 7
    def pb(o):
      srcf[pl.ds(o, 16)] = strash
      dstf[pl.ds(o, 16)] = trash16
      return o + 16
    lax.while_loop(lambda o: o < nch * CH, pb, off)

    plsc.subcore_barrier()

    def cpd(q2, b2):
      def cg(g, _):
        dstc2[q2, pl.ds(g * 16, 16)] = dstf[pl.ds(b2 + g * 16, 16)]
        return 0
      lax.fori_loop(0, CH // 16, cg, 0)

    @pl.when(nch > 0)
    def _():
      cpd(0, jnp.int32(0))
      pltpu.async_copy(x_hbm.at[srcf.at[pl.ds(0, CH)]], rows.at[0], sem_g)

    def ebody(j, _):
      p = j % 2
      q = 1 - p
      @pl.when(j > 0)
      def _():
        pltpu.make_async_copy(rows.at[q], acc.at[dstc2.at[q]], sem_s).wait()
        pltpu.make_async_copy(gc0.at[pl.ds(0, CH)], gcv, sem_c).wait()
      @pl.when(j < nch - 1)
      def _():
        cpd(q, (j + 1) * CH)
      pltpu.make_async_copy(x_hbm.at[srcf.at[pl.ds(0, CH)]], rows.at[p],
                            sem_g).wait()
      @pl.when(j < nch - 1)
      def _():
        bn = pl.multiple_of((j + 1) * CH, 8)
        pltpu.async_copy(x_hbm.at[srcf.at[pl.ds(bn, CH)]], rows.at[q], sem_g)
      pltpu.async_copy(rows.at[p], acc.at[dstc2.at[p]], sem_s, add=True)
      pltpu.async_copy(ones.at[pl.ds(0, CH)], cacc.at[dstc2.at[p]], sem_c,
                       add=True)
      return 0
    lax.fori_loop(0, nch, ebody, 0)
    @pl.when(nch > 0)
    def _():
      p = (nch - 1) % 2
      pltpu.make_async_copy(rows.at[p], acc.at[dstc2.at[p]], sem_s).wait()
      pltpu.make_async_copy(gc0.at[pl.ds(0, CH)], gcv, sem_c).wait()

    plsc.subcore_barrier()

    def gbody(qi, _):
      ob = pl.multiple_of(s * OPW + qi * CH, 8)
      def sl(g, _):
        iv = idxall[pl.ds(ob + g * 16, 16)]
        slotc[pl.ds(g * 16, 16)] = plsc.load_gather(mask, [iv])
        return 0
      lax.fori_loop(0, CH // 16, sl, 0)
      d1 = pltpu.async_copy(acc.at[slotc], rows.at[0], sem_g)
      d2 = pltpu.async_copy(cacc.at[slotc], gcv, sem_c)
      d1.wait()
      d2.wait()
      @pl.when(c == 0)
      def _():
        pltpu.sync_copy(rows.at[0], gs0.at[pl.ds(ob, CH)])
        pltpu.sync_copy(gcv, gc0.at[pl.ds(ob, CH)])
      @pl.when(c == 1)
      def _():
        pltpu.sync_copy(rows.at[0], gs1.at[pl.ds(ob, CH)])
        pltpu.sync_copy(gcv, gc1.at[pl.ds(ob, CH)])
      return 0
    lax.fori_loop(0, OCH, gbody, 0)

  return body(x, src, dst, idx_pad)


def _tc_finalize(gs0, gs1, gc0, gc1, W, b2):
  BR = 1000

  def body(g0_ref, g1_ref, c0_ref, c1_ref, w_ref, b_ref, o_ref):
    g = g0_ref[...] + g1_ref[...]
    cnt = c0_ref[...] + c1_ref[...] - 1.0
    m = g / cnt
    o_ref[...] = lax.dot_general(
        m, w_ref[...], (((1,), (1,)), ((), ())),
        preferred_element_type=jnp.float32) + b_ref[...]

  return pl.pallas_call(
      body,
      grid=(NIDX // BR,),
      in_specs=[
          pl.BlockSpec((BR, D), lambda i: (i, 0)),
          pl.BlockSpec((BR, D), lambda i: (i, 0)),
          pl.BlockSpec((BR, 1), lambda i: (i, 0)),
          pl.BlockSpec((BR, 1), lambda i: (i, 0)),
          pl.BlockSpec((D, D), lambda i: (0, 0)),
          pl.BlockSpec((1, D), lambda i: (0, 0)),
      ],
      out_specs=pl.BlockSpec((BR, D), lambda i: (i, 0)),
      out_shape=jax.ShapeDtypeStruct((NIDX, D), jnp.float32),
  )(gs0, gs1, gc0, gc1, W, b2)


def kernel(x, edge_index, idx, W, b):
  ei = edge_index.astype(jnp.int32)
  idx32 = idx.astype(jnp.int32)
  gs0, gs1, gc0, gc1 = _sc_aggregate(x, ei[0], ei[1], idx32)
  out = _tc_finalize(gs0, gs1, gc0.reshape(NP, 1), gc1.reshape(NP, 1),
                     W, b.reshape(1, D))
  return out, idx


# speedup vs baseline: 24.2051x; 1.0018x over previous
import functools

import jax
import jax.numpy as jnp
from jax import lax
from jax.experimental import pallas as pl
from jax.experimental.pallas import tpu as pltpu
from jax.experimental.pallas import tpu_sc as plsc

N = 10000
D = 128
E = 320000
NIDX = 5000

NC, NS = 2, 16
NW = NC * NS
EPW = E // NW
CH = 80
NP = NW * 160
OPW = NP // NS
OCH = OPW // CH
NA = NP + 8


def _sc_aggregate(x, src, dst, idx_pad):
  mesh = plsc.VectorSubcoreMesh(core_axis_name="c", subcore_axis_name="s",
                                num_cores=NC, num_subcores=NS)

  @functools.partial(
      pl.kernel,
      out_type=[
          jax.ShapeDtypeStruct((NP, D), jnp.float32),
          jax.ShapeDtypeStruct((NP, D), jnp.float32),
          jax.ShapeDtypeStruct((NP,), jnp.float32),
          jax.ShapeDtypeStruct((NP,), jnp.float32),
      ],
      mesh=mesh,
      compiler_params=pltpu.CompilerParams(needs_layout_passes=False),
      scratch_types=[
          pltpu.VMEM_SHARED((NA, D), jnp.float32),
          pltpu.VMEM_SHARED((NA,), jnp.float32),
          pltpu.VMEM((N,), jnp.int32),
          pltpu.VMEM((NP,), jnp.int32),
          pltpu.VMEM((EPW,), jnp.int32),
          pltpu.VMEM((EPW,), jnp.int32),
          pltpu.VMEM((EPW + CH,), jnp.int32),
          pltpu.VMEM((EPW + CH,), jnp.int32),
          pltpu.VMEM((2, CH), jnp.int32),
          pltpu.VMEM((2, CH, D), jnp.float32),
          pltpu.VMEM((1008,), jnp.float32),
          pltpu.VMEM((16, D), jnp.float32),
          pltpu.VMEM((CH,), jnp.int32),
          pltpu.VMEM((CH,), jnp.int32),
          pltpu.VMEM((CH,), jnp.float32),
          pltpu.VMEM((2, CH), jnp.float32),
          pltpu.SemaphoreType.DMA,
          pltpu.SemaphoreType.DMA,
          pltpu.SemaphoreType.DMA,
          pltpu.SemaphoreType.DMA,
      ],
  )
  def body(x_hbm, src_hbm, dst_hbm, idx_hbm, gs0, gs1, gc0, gc1,
           acc, cacc, mask, idxall, srcall, dstall, srcf, dstf, dstc2, rows,
           ones, zrow, idxc, slotc, gcv, gcv2, sem_w, sem_g, sem_s, sem_c):
    c = lax.axis_index("c")
    s = lax.axis_index("s")
    wid = c * NS + s
    lane = lax.iota(jnp.int32, 16)
    trash16 = NP + (lane & 7)

    ebase = pl.multiple_of(wid * EPW, 8)
    pltpu.async_copy(src_hbm.at[pl.ds(ebase, EPW)], srcall, sem_g)
    pltpu.async_copy(dst_hbm.at[pl.ds(ebase, EPW)], dstall, sem_g)
    pltpu.async_copy(idx_hbm.at[pl.ds(0, NIDX - 8)], idxall.at[pl.ds(0, NIDX - 8)], sem_g)
    pltpu.async_copy(idx_hbm.at[pl.ds(NIDX - 16, 16)], idxall.at[pl.ds(NIDX - 16, 16)], sem_g)

    def fill_ones(i, _):
      ones[pl.ds(i * 16, 16)] = jnp.full((16,), 1.0, jnp.float32)
      return 0
    lax.fori_loop(0, 63, fill_ones, 0)

    def fill_zr(i, _):
      zrow[i // 8, pl.ds((i % 8) * 16, 16)] = jnp.zeros((16,), jnp.float32)
      return 0
    lax.fori_loop(0, 16 * 8, fill_zr, 0)

    def zm(i, _):
      mask[pl.ds(i * 16, 16)] = trash16
      return 0
    lax.fori_loop(0, N // 16, zm, 0)

    @pl.when(s < 5)
    def _():
      pltpu.sync_copy(ones.at[pl.ds(0, 1008)], cacc.at[pl.ds(s * 1008, 1008)])
    @pl.when(s == 5)
    def _():
      pltpu.sync_copy(ones.at[pl.ds(0, NA - 5 * 1008)],
                      cacc.at[pl.ds(5 * 1008, NA - 5 * 1008)])

    zbase = s * OPW + (1 - c) * (OPW // 2)
    def zc(i, _):
      pltpu.sync_copy(zrow, acc.at[pl.ds(zbase + i * 16, 16)])
      return 0
    lax.fori_loop(0, (OPW // 2) // 16, zc, 0)
    @pl.when(s == NS - 1)
    def _():
      pltpu.sync_copy(zrow.at[pl.ds(0, 8)], acc.at[pl.ds(NP, 8)])

    pltpu.make_async_copy(src_hbm.at[pl.ds(ebase, EPW)], srcall, sem_g).wait()
    pltpu.make_async_copy(dst_hbm.at[pl.ds(ebase, EPW)], dstall, sem_g).wait()
    pltpu.make_async_copy(idx_hbm.at[pl.ds(0, NIDX - 8)], idxall.at[pl.ds(0, NIDX - 8)], sem_g).wait()
    pltpu.make_async_copy(idx_hbm.at[pl.ds(NIDX - 16, 16)], idxall.at[pl.ds(NIDX - 16, 16)], sem_g).wait()
    tail = idxall[pl.ds(NIDX - 8, 16)]
    idxall[pl.ds(NIDX - 8, 16)] = jnp.where(lane < 8, tail, 0)
    def zpad(k, _):
      idxall[pl.ds(NIDX + 8 + k * 16, 16)] = jnp.zeros((16,), jnp.int32)
      return 0
    lax.fori_loop(0, (NP - NIDX - 8) // 16, zpad, 0)

    def bm(g, _):
      iv = idxall[pl.ds(g * 16, 16)]
      plsc.store_scatter(mask, [iv], g * 16 + lane)
      return 0
    lax.fori_loop(0, NP // 16, bm, 0)

    def xi(qi, _):
      ob = pl.multiple_of(s * OPW + c * (OPW // 2) + qi * CH, 8)
      def cpi(g, _):
        idxc[pl.ds(g * 16, 16)] = idxall[pl.ds(ob + g * 16, 16)]
        return 0
      lax.fori_loop(0, CH // 16, cpi, 0)
      pltpu.async_copy(x_hbm.at[idxc], rows.at[0], sem_g).wait()
      pltpu.sync_copy(rows.at[0], acc.at[pl.ds(ob, CH)])
      return 0
    lax.fori_loop(0, (OPW // 2) // CH, xi, 0)

    def fb1(g, off):
      sval = srcall[pl.ds(g * 16, 16)]
      dval = dstall[pl.ds(g * 16, 16)]
      mv = plsc.load_gather(mask, [dval])
      keep = mv < NP
      plsc.store_compressed(srcf.at[pl.ds(off, 16)], sval, mask=keep)
      plsc.store_compressed(dstf.at[pl.ds(off, 16)], mv, mask=keep)
      return off + plsc.all_reduce_population_count(keep)[0]
    def fb2(h, off):
      return fb1(2 * h + 1, fb1(2 * h, off))
    off = lax.fori_loop(0, (EPW // 16) // 2, fb2, jnp.int32(0))
    off = fb1(EPW // 16 - 1, off)

    nch = (off + (CH - 1)) // CH
    strash = lane & 7
    def pb(o):
      srcf[pl.ds(o, 16)] = strash
      dstf[pl.ds(o, 16)] = trash16
      return o + 16
    lax.while_loop(lambda o: o < nch * CH, pb, off)

    plsc.subcore_barrier()

    def cpd(q2, b2):
      def cg(g, _):
        dstc2[q2, pl.ds(g * 16, 16)] = dstf[pl.ds(b2 + g * 16, 16)]
        return 0
      lax.fori_loop(0, CH // 16, cg, 0)

    @pl.when(nch > 0)
    def _():
      cpd(0, jnp.int32(0))
      pltpu.async_copy(x_hbm.at[srcf.at[pl.ds(0, CH)]], rows.at[0], sem_g)

    def ebody(j, _):
      p = j % 2
      q = 1 - p
      @pl.when(j > 0)
      def _():
        pltpu.make_async_copy(rows.at[q], acc.at[dstc2.at[q]], sem_s).wait()
        pltpu.make_async_copy(gc0.at[pl.ds(0, CH)], gcv, sem_c).wait()
      @pl.when(j < nch - 1)
      def _():
        cpd(q, (j + 1) * CH)
      pltpu.make_async_copy(x_hbm.at[srcf.at[pl.ds(0, CH)]], rows.at[p],
                            sem_g).wait()
      @pl.when(j < nch - 1)
      def _():
        bn = pl.multiple_of((j + 1) * CH, 8)
        pltpu.async_copy(x_hbm.at[srcf.at[pl.ds(bn, CH)]], rows.at[q], sem_g)
      pltpu.async_copy(rows.at[p], acc.at[dstc2.at[p]], sem_s, add=True)
      pltpu.async_copy(ones.at[pl.ds(0, CH)], cacc.at[dstc2.at[p]], sem_c,
                       add=True)
      return 0
    lax.fori_loop(0, nch, ebody, 0)
    @pl.when(nch > 0)
    def _():
      p = (nch - 1) % 2
      pltpu.make_async_copy(rows.at[p], acc.at[dstc2.at[p]], sem_s).wait()
      pltpu.make_async_copy(gc0.at[pl.ds(0, CH)], gcv, sem_c).wait()

    plsc.subcore_barrier()

    def gbody(qi, _):
      p = qi % 2
      ob = pl.multiple_of(s * OPW + qi * CH, 8)
      obp = pl.multiple_of(s * OPW + (qi - 1) * CH, 8)
      @pl.when(qi > 0)
      def _():
        pltpu.make_async_copy(rows.at[1 - p], gs0.at[pl.ds(ob, CH)],
                              sem_w).wait()
        pltpu.make_async_copy(gcv2.at[1 - p], gc0.at[pl.ds(ob, CH)],
                              sem_w).wait()
      def sl(g, _):
        iv = idxall[pl.ds(ob + g * 16, 16)]
        slotc[pl.ds(g * 16, 16)] = plsc.load_gather(mask, [iv])
        return 0
      lax.fori_loop(0, CH // 16, sl, 0)
      d1 = pltpu.async_copy(acc.at[slotc], rows.at[p], sem_g)
      d2 = pltpu.async_copy(cacc.at[slotc], gcv2.at[p], sem_c)
      d1.wait()
      d2.wait()
      @pl.when(c == 0)
      def _():
        pltpu.async_copy(rows.at[p], gs0.at[pl.ds(ob, CH)], sem_w)
        pltpu.async_copy(gcv2.at[p], gc0.at[pl.ds(ob, CH)], sem_w)
      @pl.when(c == 1)
      def _():
        pltpu.async_copy(rows.at[p], gs1.at[pl.ds(ob, CH)], sem_w)
        pltpu.async_copy(gcv2.at[p], gc1.at[pl.ds(ob, CH)], sem_w)
      return 0
    lax.fori_loop(0, OCH, gbody, 0)
    pf = (OCH - 1) % 2
    pltpu.make_async_copy(rows.at[pf], gs0.at[pl.ds(0, CH)], sem_w).wait()
    pltpu.make_async_copy(gcv2.at[pf], gc0.at[pl.ds(0, CH)], sem_w).wait()

  return body(x, src, dst, idx_pad)


def _tc_finalize(gs0, gs1, gc0, gc1, W, b2):
  BR = 1000

  def body(g0_ref, g1_ref, c0_ref, c1_ref, w_ref, b_ref, o_ref):
    g = g0_ref[...] + g1_ref[...]
    cnt = c0_ref[...] + c1_ref[...] - 1.0
    m = g / cnt
    o_ref[...] = lax.dot_general(
        m, w_ref[...], (((1,), (1,)), ((), ())),
        preferred_element_type=jnp.float32) + b_ref[...]

  return pl.pallas_call(
      body,
      grid=(NIDX // BR,),
      in_specs=[
          pl.BlockSpec((BR, D), lambda i: (i, 0)),
          pl.BlockSpec((BR, D), lambda i: (i, 0)),
          pl.BlockSpec((BR, 1), lambda i: (i, 0)),
          pl.BlockSpec((BR, 1), lambda i: (i, 0)),
          pl.BlockSpec((D, D), lambda i: (0, 0)),
          pl.BlockSpec((1, D), lambda i: (0, 0)),
      ],
      out_specs=pl.BlockSpec((BR, D), lambda i: (i, 0)),
      out_shape=jax.ShapeDtypeStruct((NIDX, D), jnp.float32),
  )(gs0, gs1, gc0, gc1, W, b2)


def kernel(x, edge_index, idx, W, b):
  ei = edge_index.astype(jnp.int32)
  idx32 = idx.astype(jnp.int32)
  gs0, gs1, gc0, gc1 = _sc_aggregate(x, ei[0], ei[1], idx32)
  out = _tc_finalize(gs0, gs1, gc0.reshape(NP, 1), gc1.reshape(NP, 1),
                     W, b.reshape(1, D))
  return out, idx
